# baseline (speedup 1.0000x reference)
import jax
import jax.numpy as jnp
from jax import lax
from jax.experimental import pallas as pl
from jax.experimental.pallas import tpu as pltpu
from jax.experimental.pallas import tpu_sc as plsc

N = 10000
E = 160000
D = 256
KV = D // 16
NC, NS = 2, 16
NW = NC * NS
DELTA = 320
NPAD = NW * DELTA
BATCH = 1600
NB = E // BATCH
VPB = BATCH // 16
GCH = 32
GU = 8
NEG = 0.2


def _mm_body(x_ref, w_ref, o_ref):
    o_ref[...] = jnp.dot(x_ref[...], w_ref[...],
                         preferred_element_type=jnp.float32)


def _matmul(x, w):
    m, k = x.shape
    n = w.shape[1]
    bm = 1024
    return pl.pallas_call(
        _mm_body,
        grid=(m // bm,),
        in_specs=[
            pl.BlockSpec((bm, k), lambda i: (i, 0)),
            pl.BlockSpec((k, n), lambda i: (0, 0)),
        ],
        out_specs=pl.BlockSpec((bm, n), lambda i: (i, 0)),
        out_shape=jax.ShapeDtypeStruct((m, n), jnp.float32),
    )(x, w)


def _l1_body(h_hbm, src_hbm, dst_hbm, a1_hbm, w2_hbm, s_hbm,
             acc, mflat, dflat, a1v, w2v, srcbA, srcbB, dstbA, dstbB,
             osrc, odst, repv, hsA, hsB, hdA, hdB, sloc,
             sembA, sembB, semcA, semcB):
    _srcb = [srcbA, srcbB]
    _dstb = [dstbA, dstbB]
    _hs = [hsA, hsB]
    _hd = [hdA, hdB]
    _semb = [sembA, sembB]
    _semc = [semcA, semcB]
    wid = lax.axis_index("s") * NC + lax.axis_index("c")
    base = wid * DELTA
    zf = jnp.zeros((16,), jnp.float32)
    zi = jnp.zeros((16,), jnp.int32)
    lane0 = lax.iota(jnp.int32, 16) == 0

    pltpu.sync_copy(a1_hbm, a1v)
    pltpu.sync_copy(w2_hbm, w2v)

    def _z_rows(j, _):
        for k in range(KV):
            acc[j, pl.ds(k * 16, 16)] = zf
        return 0
    lax.fori_loop(0, DELTA, _z_rows, 0)

    def _z_rep(v, _):
        repv[pl.ds(v * 16, 16)] = zi
        dflat[pl.ds(v * 16, 16)] = zf
        return 0
    lax.fori_loop(0, DELTA // 16, _z_rep, 0)

    def _scan_batches(body):
        def _issue(b, par):
            sm = _semb[par]
            pltpu.async_copy(src_hbm.at[pl.ds(b * BATCH, BATCH)],
                             _srcb[par], sm)
            pltpu.async_copy(dst_hbm.at[pl.ds(b * BATCH, BATCH)],
                             _dstb[par], sm)

        def _wait(par):
            sm = _semb[par]
            pltpu.make_async_copy(src_hbm.at[pl.ds(0, BATCH)],
                                  _srcb[par], sm).wait()
            pltpu.make_async_copy(dst_hbm.at[pl.ds(0, BATCH)],
                                  _dstb[par], sm).wait()

        _issue(0, 0)

        def _pair(bp, _):
            for par in range(2):
                b = 2 * bp + par
                nxt = b + 1

                @pl.when(nxt < NB)
                def _():
                    _issue(nxt, 1 - par)
                _wait(par)
                body(_srcb[par], _dstb[par], b)
            return 0
        lax.fori_loop(0, NB // 2, _pair, 0)

    def _p1(sr, dr, b):
        def _v(v, _):
            sv = sr[pl.ds(v * 16, 16)]
            dv = dr[pl.ds(v * 16, 16)]
            dl = dv - base
            msk = (dl >= 0) & (dl < DELTA)
            plsc.store_scatter(repv, [dl], sv, mask=msk)
            return 0
        lax.fori_loop(0, VPB, _v, 0)
    _scan_batches(_p1)

    a1list = [a1v[pl.ds(k * 16, 16)] for k in range(KV)]

    def _escore(hsb, hdb, i):
        ev = [zf, zf, zf, zf]
        hss = []
        for k in range(KV):
            hk = hsb[i, pl.ds(k * 16, 16)]
            t = hk + hdb[i, pl.ds(k * 16, 16)]
            t = jnp.maximum(t, t * NEG)
            ev[k % 4] = ev[k % 4] + a1list[k] * t
            hss.append(hk)
        return (ev[0] + ev[1]) + (ev[2] + ev[3]), hss

    def _p1b(c, _):
        pltpu.sync_copy(h_hbm.at[pl.ds(base + c * GCH, GCH)], hdA)
        pltpu.async_copy(h_hbm.at[repv.at[pl.ds(c * GCH, GCH)]],
                         hsA, semcA).wait()

        def _i(i, _):
            ev, _hss = _escore(hsA, hdA, i)
            tot = jnp.sum(ev)
            plsc.store_scatter(mflat, [jnp.full((16,), c * GCH + i, jnp.int32)],
                               jnp.full((16,), tot, jnp.float32), mask=lane0)
            return 0
        lax.fori_loop(0, GCH, _i, 0)
        return 0
    lax.fori_loop(0, DELTA // GCH, _p1b, 0)

    def _issue_ch(c, par):
        sm = _semc[par]
        pltpu.async_copy(h_hbm.at[osrc.at[pl.ds(c * GCH, GCH)]],
                         _hs[par], sm)
        pltpu.async_copy(h_hbm.at[odst.at[pl.ds(c * GCH, GCH)]],
                         _hd[par], sm)

    def _wait_ch(par):
        sm = _semc[par]
        pltpu.make_async_copy(h_hbm.at[pl.ds(0, GCH)], _hs[par], sm).wait()
        pltpu.make_async_copy(h_hbm.at[pl.ds(0, GCH)], _hd[par], sm).wait()

    def _p2(sr, dr, b):
        def _cmp(v, off):
            sv = sr[pl.ds(v * 16, 16)]
            dv = dr[pl.ds(v * 16, 16)]
            dl = dv - base
            msk = (dl >= 0) & (dl < DELTA)
            mi = msk.astype(jnp.int32)
            pos = off + plsc.cumsum(mi) - 1
            plsc.store_scatter(osrc, [pos], sv, mask=msk)
            plsc.store_scatter(odst, [pos], dv, mask=msk)
            return off + jnp.sum(mi)
        cnt = lax.fori_loop(0, VPB, _cmp, jnp.int32(0))

        iota16 = lax.iota(jnp.int32, 16)
        for t in range(3):
            pos = cnt + t * 16 + iota16
            pmsk = pos < (BATCH + 64)
            plsc.store_scatter(odst, [pos],
                               jnp.full((16,), base + DELTA, jnp.int32),
                               mask=pmsk)
            plsc.store_scatter(osrc, [pos], zi, mask=pmsk)

        nch = (cnt + (GCH - 1)) // GCH

        @pl.when(nch > 0)
        def _():
            _issue_ch(0, 0)

        def _proc(c, par):
            _wait_ch(par)

            def _grp(g, _):
                dlv = odst[pl.ds(c * GCH + g * GU, 16)] - base
                for u in range(GU):
                    i = g * GU + u
                    dl = dlv[u]
                    idxv = jnp.full((16,), dl, jnp.int32)
                    ev, hss = _escore(_hs[par], _hd[par], i)
                    tot = jnp.sum(ev)
                    mv = plsc.load_gather(mflat, [idxv])
                    w = jnp.exp(jnp.full((16,), tot, jnp.float32) - mv)
                    plsc.addupdate_scatter(dflat, [idxv], w, mask=lane0)
                    for k in range(KV):
                        plsc.addupdate(acc.at[dl, pl.ds(k * 16, 16)],
                                       w * hss[k])
            lax.fori_loop(0, GCH // GU, lambda g, _: (_grp(g, _), 0)[1], 0)

        def _chpair(cp, _):
            for par in range(2):
                c = 2 * cp + par

                @pl.when(c < nch)
                def _():
                    @pl.when(c + 1 < nch)
                    def _():
                        _issue_ch(c + 1, 1 - par)
                    _proc(c, par)
            return 0
        lax.fori_loop(0, (nch + 1) // 2, _chpair, 0)
    _scan_batches(_p2)

    def _p3(jj, _):
        dv = plsc.load_gather(dflat, [jnp.full((16,), jj, jnp.int32)]) + 1e-16
        sv2 = zf
        for k in range(KV):
            t = acc[jj, pl.ds(k * 16, 16)] / dv
            t = jnp.maximum(t, 0.0)
            sv2 = sv2 + t * w2v[pl.ds(k * 16, 16)]
        plsc.store_scatter(sloc, [jnp.full((16,), jj, jnp.int32)],
                           jnp.full((16,), jnp.sum(sv2), jnp.float32),
                           mask=lane0)
        return 0
    lax.fori_loop(0, DELTA, _p3, 0)
    pltpu.sync_copy(sloc, s_hbm.at[pl.ds(base, DELTA)])


def _l2_body(s_hbm, src_hbm, dst_hbm, p_hbm, o_hbm,
             sfull, mloc, dloc, nloc, repv, srcbA, srcbB, dstbA, dstbB,
             pv, outloc, sembA, sembB):
    _srcb = [srcbA, srcbB]
    _dstb = [dstbA, dstbB]
    _semb = [sembA, sembB]
    wid = lax.axis_index("s") * NC + lax.axis_index("c")
    base = wid * DELTA
    zf = jnp.zeros((16,), jnp.float32)
    zi = jnp.zeros((16,), jnp.int32)

    pltpu.sync_copy(s_hbm, sfull)
    pltpu.sync_copy(p_hbm, pv)
    a2v = jnp.full((16,), pv[...][0], jnp.float32)

    def _z(v, _):
        mloc[pl.ds(v * 16, 16)] = zf
        dloc[pl.ds(v * 16, 16)] = zf
        nloc[pl.ds(v * 16, 16)] = zf
        repv[pl.ds(v * 16, 16)] = zi
        return 0
    lax.fori_loop(0, DELTA // 16, _z, 0)

    def _scan_batches(body):
        def _issue(b, par):
            sm = _semb[par]
            pltpu.async_copy(src_hbm.at[pl.ds(b * BATCH, BATCH)],
                             _srcb[par], sm)
            pltpu.async_copy(dst_hbm.at[pl.ds(b * BATCH, BATCH)],
                             _dstb[par], sm)

        def _wait(par):
            sm = _semb[par]
            pltpu.make_async_copy(src_hbm.at[pl.ds(0, BATCH)],
                                  _srcb[par], sm).wait()
            pltpu.make_async_copy(dst_hbm.at[pl.ds(0, BATCH)],
                                  _dstb[par], sm).wait()

        _issue(0, 0)

        def _pair(bp, _):
            for par in range(2):
                b = 2 * bp + par
                nxt = b + 1

                @pl.when(nxt < NB)
                def _():
                    _issue(nxt, 1 - par)
                _wait(par)
                body(_srcb[par], _dstb[par], b)
            return 0
        lax.fori_loop(0, NB // 2, _pair, 0)

    def _p1(sr, dr, b):
        def _v(v, _):
            for u in range(5):
                vv = v * 5 + u
                sv = sr[pl.ds(vv * 16, 16)]
                dv = dr[pl.ds(vv * 16, 16)]
                dl = dv - base
                msk = (dl >= 0) & (dl < DELTA)
                plsc.store_scatter(repv, [dl], sv, mask=msk)
            return 0
        lax.fori_loop(0, VPB // 5, _v, 0)
    _scan_batches(_p1)

    def _p1b(j, _):
        rs = repv[pl.ds(j * 16, 16)]
        ss = plsc.load_gather(sfull, [rs])
        sd = sfull[pl.ds(base + j * 16, 16)]
        t = ss + sd
        e = a2v * jnp.maximum(t, t * NEG)
        mloc[pl.ds(j * 16, 16)] = e
        return 0
    lax.fori_loop(0, DELTA // 16, _p1b, 0)

    def _p2(sr, dr, b):
        def _v(v, _):
            for u in range(5):
                vv = v * 5 + u
                sv = sr[pl.ds(vv * 16, 16)]
                dv = dr[pl.ds(vv * 16, 16)]
                ss = plsc.load_gather(sfull, [sv])
                sd = plsc.load_gather(sfull, [dv])
                t = ss + sd
                e = a2v * jnp.maximum(t, t * NEG)
                dl = dv - base
                msk = (dl >= 0) & (dl < DELTA)
                mv = plsc.load_gather(mloc, [dl], mask=msk)
                w = jnp.exp(e - mv)
                plsc.addupdate_scatter(dloc, [dl], w, mask=msk)
                plsc.addupdate_scatter(nloc, [dl], w * ss, mask=msk)
            return 0
        lax.fori_loop(0, VPB // 5, _v, 0)
    _scan_batches(_p2)

    def _p3(j, _):
        d = dloc[pl.ds(j * 16, 16)] + 1e-16
        n = nloc[pl.ds(j * 16, 16)]
        o = n / d
        outloc[pl.ds(j * 16, 16)] = 1.0 / (1.0 + jnp.exp(-o))
        return 0
    lax.fori_loop(0, DELTA // 16, _p3, 0)
    pltpu.sync_copy(outloc, o_hbm.at[pl.ds(base, DELTA)])


def _sc_mesh():
    return plsc.VectorSubcoreMesh(core_axis_name="c", subcore_axis_name="s",
                                  num_cores=NC, num_subcores=NS)


def _layer1(h, src, dst, a1, w2col):
    return pl.kernel(
        _l1_body,
        out_type=jax.ShapeDtypeStruct((NPAD,), jnp.float32),
        mesh=_sc_mesh(),
        compiler_params=pltpu.CompilerParams(needs_layout_passes=False),
        scratch_types=[
            pltpu.VMEM((DELTA + 1, D), jnp.float32),
            pltpu.VMEM((DELTA + 16,), jnp.float32),
            pltpu.VMEM((DELTA + 16,), jnp.float32),
            pltpu.VMEM((D,), jnp.float32),
            pltpu.VMEM((D,), jnp.float32),
            pltpu.VMEM((BATCH,), jnp.int32),
            pltpu.VMEM((BATCH,), jnp.int32),
            pltpu.VMEM((BATCH,), jnp.int32),
            pltpu.VMEM((BATCH,), jnp.int32),
            pltpu.VMEM((BATCH + 64,), jnp.int32),
            pltpu.VMEM((BATCH + 64,), jnp.int32),
            pltpu.VMEM((DELTA,), jnp.int32),
            pltpu.VMEM((GCH, D), jnp.float32),
            pltpu.VMEM((GCH, D), jnp.float32),
            pltpu.VMEM((GCH, D), jnp.float32),
            pltpu.VMEM((GCH, D), jnp.float32),
            pltpu.VMEM((DELTA,), jnp.float32),
            pltpu.SemaphoreType.DMA,
            pltpu.SemaphoreType.DMA,
            pltpu.SemaphoreType.DMA,
            pltpu.SemaphoreType.DMA,
        ],
    )(h, src, dst, a1, w2col)


def _layer2(s, src, dst, params):
    return pl.kernel(
        _l2_body,
        out_type=jax.ShapeDtypeStruct((NPAD,), jnp.float32),
        mesh=_sc_mesh(),
        compiler_params=pltpu.CompilerParams(needs_layout_passes=False),
        scratch_types=[
            pltpu.VMEM((NPAD,), jnp.float32),
            pltpu.VMEM((DELTA,), jnp.float32),
            pltpu.VMEM((DELTA,), jnp.float32),
            pltpu.VMEM((DELTA,), jnp.float32),
            pltpu.VMEM((DELTA,), jnp.int32),
            pltpu.VMEM((BATCH,), jnp.int32),
            pltpu.VMEM((BATCH,), jnp.int32),
            pltpu.VMEM((BATCH,), jnp.int32),
            pltpu.VMEM((BATCH,), jnp.int32),
            pltpu.VMEM((16,), jnp.float32),
            pltpu.VMEM((DELTA,), jnp.float32),
            pltpu.SemaphoreType.DMA,
            pltpu.SemaphoreType.DMA,
        ],
    )(s, src, dst, params)


def kernel(feature, edge_index, W1, a1, W2, a2):
    src = edge_index[0].astype(jnp.int32)
    dst = edge_index[1].astype(jnp.int32)
    xpad = jnp.pad(feature, ((0, NPAD - N), (0, 0)))
    h = _matmul(xpad, W1)
    s = _layer1(h, src, dst, a1, W2[:, 0])
    params = jnp.pad(a2, (0, 15))
    out = _layer2(s, src, dst, params)
    return out[:N][:, None]


# speedup vs baseline: 1.6770x; 1.6770x over previous
import jax
import jax.numpy as jnp
from jax import lax
from jax.experimental import pallas as pl
from jax.experimental.pallas import tpu as pltpu
from jax.experimental.pallas import tpu_sc as plsc

N = 10000
E = 160000
D = 256
KV = D // 16
NC, NS = 2, 16
NW = NC * NS
DELTA = 320
NPAD = NW * DELTA
BATCH = 1600
NB = E // BATCH
VPB = BATCH // 16
GCH = 32
GU = 8
NEG = 0.2


def _mm_body(x_ref, w_ref, o_ref):
    o_ref[...] = jnp.dot(x_ref[...], w_ref[...],
                         preferred_element_type=jnp.float32)


def _matmul(x, w):
    m, k = x.shape
    n = w.shape[1]
    bm = 1024
    return pl.pallas_call(
        _mm_body,
        grid=(m // bm,),
        in_specs=[
            pl.BlockSpec((bm, k), lambda i: (i, 0)),
            pl.BlockSpec((k, n), lambda i: (0, 0)),
        ],
        out_specs=pl.BlockSpec((bm, n), lambda i: (i, 0)),
        out_shape=jax.ShapeDtypeStruct((m, n), jnp.float32),
    )(x, w)


def _l1_body(h_hbm, src_hbm, dst_hbm, a1_hbm, w2_hbm, s_hbm,
             acc, mflat, dflat, a1v, w2v, srcbA, srcbB, dstbA, dstbB,
             osrc, odst, repv, hsA, hsB, hdA, hdB, sloc,
             sembA, sembB, semcA, semcB):
    _srcb = [srcbA, srcbB]
    _dstb = [dstbA, dstbB]
    _hs = [hsA, hsB]
    _hd = [hdA, hdB]
    _semb = [sembA, sembB]
    _semc = [semcA, semcB]
    wid = lax.axis_index("s") * NC + lax.axis_index("c")
    base = wid * DELTA
    zf = jnp.zeros((16,), jnp.float32)
    zi = jnp.zeros((16,), jnp.int32)
    lane0 = lax.iota(jnp.int32, 16) == 0

    pltpu.sync_copy(a1_hbm, a1v)
    pltpu.sync_copy(w2_hbm, w2v)

    def _z_rows(j, _):
        for k in range(KV):
            acc[j, pl.ds(k * 16, 16)] = zf
        return 0
    lax.fori_loop(0, DELTA, _z_rows, 0)

    def _z_rep(v, _):
        repv[pl.ds(v * 16, 16)] = zi
        dflat[pl.ds(v * 16, 16)] = zf
        return 0
    lax.fori_loop(0, DELTA // 16, _z_rep, 0)

    def _scan_batches(body, init):
        def _issue(b, par):
            sm = _semb[par]
            pltpu.async_copy(src_hbm.at[pl.ds(b * BATCH, BATCH)],
                             _srcb[par], sm)
            pltpu.async_copy(dst_hbm.at[pl.ds(b * BATCH, BATCH)],
                             _dstb[par], sm)

        def _wait(par):
            sm = _semb[par]
            pltpu.make_async_copy(src_hbm.at[pl.ds(0, BATCH)],
                                  _srcb[par], sm).wait()
            pltpu.make_async_copy(dst_hbm.at[pl.ds(0, BATCH)],
                                  _dstb[par], sm).wait()

        _issue(0, 0)

        def _pair(bp, carry):
            for par in range(2):
                b = 2 * bp + par
                nxt = b + 1

                @pl.when(nxt < NB)
                def _():
                    _issue(nxt, 1 - par)
                _wait(par)
                carry = body(_srcb[par], _dstb[par], b, carry)
            return carry
        return lax.fori_loop(0, NB // 2, _pair, init)

    def _p1(sr, dr, b, carry):
        def _v(v, _):
            sv = sr[pl.ds(v * 16, 16)]
            dv = dr[pl.ds(v * 16, 16)]
            dl = dv - base
            msk = (dl >= 0) & (dl < DELTA)
            plsc.store_scatter(repv, [dl], sv, mask=msk)
            return 0
        lax.fori_loop(0, VPB, _v, 0)
        return carry
    _scan_batches(_p1, jnp.int32(0))

    a1list = [a1v[pl.ds(k * 16, 16)] for k in range(KV)]

    def _escore(hsb, hdb, i):
        ev = [zf, zf, zf, zf]
        for k in range(KV):
            hk = hsb[i, pl.ds(k * 16, 16)]
            t = hk + hdb[i, pl.ds(k * 16, 16)]
            t = jnp.maximum(t, t * NEG)
            ev[k % 4] = ev[k % 4] + a1list[k] * t
        return (ev[0] + ev[1]) + (ev[2] + ev[3])

    def _p1b(c, _):
        pltpu.sync_copy(h_hbm.at[pl.ds(base + c * GCH, GCH)], hdA)
        pltpu.async_copy(h_hbm.at[repv.at[pl.ds(c * GCH, GCH)]],
                         hsA, semcA).wait()

        def _i(i, _):
            tot = jnp.sum(_escore(hsA, hdA, i))
            plsc.store_scatter(mflat, [jnp.full((16,), c * GCH + i, jnp.int32)],
                               jnp.full((16,), tot, jnp.float32), mask=lane0)
            return 0
        lax.fori_loop(0, GCH, _i, 0)
        return 0
    lax.fori_loop(0, DELTA // GCH, _p1b, 0)

    def _issue_ch(c, par):
        sm = _semc[par]
        pltpu.async_copy(h_hbm.at[osrc.at[pl.ds(c * GCH, GCH)]],
                         _hs[par], sm)
        pltpu.async_copy(h_hbm.at[odst.at[pl.ds(c * GCH, GCH)]],
                         _hd[par], sm)

    def _wait_ch(par):
        sm = _semc[par]
        pltpu.make_async_copy(h_hbm.at[pl.ds(0, GCH)], _hs[par], sm).wait()
        pltpu.make_async_copy(h_hbm.at[pl.ds(0, GCH)], _hd[par], sm).wait()

    def _proc(c, par):
        _wait_ch(par)

        def _grp(g, _):
            dlv = odst[pl.ds(c * GCH + g * GU, 16)] - base
            for u in range(GU):
                i = g * GU + u
                dl = dlv[u]
                idxv = jnp.full((16,), dl, jnp.int32)
                tot = jnp.sum(_escore(_hs[par], _hd[par], i))
                mv = plsc.load_gather(mflat, [idxv])
                w = jnp.exp(jnp.full((16,), tot, jnp.float32) - mv)
                plsc.addupdate_scatter(dflat, [idxv], w, mask=lane0)
                for k in range(KV):
                    plsc.addupdate(acc.at[dl, pl.ds(k * 16, 16)],
                                   w * _hs[par][i, pl.ds(k * 16, 16)])
            return 0
        lax.fori_loop(0, GCH // GU, _grp, 0)

    def _run_chunks(nch):
        def _chpair(cp, _):
            for par in range(2):
                c = 2 * cp + par

                @pl.when(c < nch)
                def _():
                    @pl.when(c + 1 < nch)
                    def _():
                        _issue_ch(c + 1, 1 - par)
                    _proc(c, par)
            return 0
        lax.fori_loop(0, (nch + 1) // 2, _chpair, 0)

    def _p2(sr, dr, b, lo):
        def _cmp(v, off):
            sv = sr[pl.ds(v * 16, 16)]
            dv = dr[pl.ds(v * 16, 16)]
            dl = dv - base
            msk = (dl >= 0) & (dl < DELTA)
            mi = msk.astype(jnp.int32)
            pos = off + plsc.cumsum(mi) - 1
            plsc.store_scatter(osrc, [pos], sv, mask=msk)
            plsc.store_scatter(odst, [pos], dv, mask=msk)
            return off + jnp.sum(mi)
        cnt = lax.fori_loop(0, VPB, _cmp, lo)

        nfull = cnt // GCH

        @pl.when(nfull > 0)
        def _():
            _issue_ch(0, 0)
            _run_chunks(nfull)
            tail = nfull * GCH
            d0 = odst[pl.ds(tail, 16)]
            d1 = odst[pl.ds(tail + 16, 16)]
            s0 = osrc[pl.ds(tail, 16)]
            s1 = osrc[pl.ds(tail + 16, 16)]
            odst[pl.ds(0, 16)] = d0
            odst[pl.ds(16, 16)] = d1
            osrc[pl.ds(0, 16)] = s0
            osrc[pl.ds(16, 16)] = s1
        return cnt - nfull * GCH

    rem = _scan_batches(_p2, jnp.int32(0))

    @pl.when(rem > 0)
    def _():
        iota16 = lax.iota(jnp.int32, 16)
        for t in range(2):
            pos = rem + t * 16 + iota16
            pmsk = pos < (BATCH + 64)
            plsc.store_scatter(odst, [pos],
                               jnp.full((16,), base + DELTA, jnp.int32),
                               mask=pmsk)
            plsc.store_scatter(osrc, [pos], zi, mask=pmsk)
        _issue_ch(0, 0)
        _proc(0, 0)

    def _p3(jj, _):
        dv = plsc.load_gather(dflat, [jnp.full((16,), jj, jnp.int32)]) + 1e-16
        sv2 = zf
        for k in range(KV):
            t = acc[jj, pl.ds(k * 16, 16)] / dv
            t = jnp.maximum(t, 0.0)
            sv2 = sv2 + t * w2v[pl.ds(k * 16, 16)]
        plsc.store_scatter(sloc, [jnp.full((16,), jj, jnp.int32)],
                           jnp.full((16,), jnp.sum(sv2), jnp.float32),
                           mask=lane0)
        return 0
    lax.fori_loop(0, DELTA, _p3, 0)
    pltpu.sync_copy(sloc, s_hbm.at[pl.ds(base, DELTA)])


def _l2_body(s_hbm, src_hbm, dst_hbm, p_hbm, o_hbm,
             sfull, mloc, dloc, nloc, repv, srcbA, srcbB, dstbA, dstbB,
             pv, outloc, sembA, sembB):
    _srcb = [srcbA, srcbB]
    _dstb = [dstbA, dstbB]
    _semb = [sembA, sembB]
    wid = lax.axis_index("s") * NC + lax.axis_index("c")
    base = wid * DELTA
    zf = jnp.zeros((16,), jnp.float32)
    zi = jnp.zeros((16,), jnp.int32)

    pltpu.sync_copy(s_hbm, sfull)
    pltpu.sync_copy(p_hbm, pv)
    a2v = jnp.full((16,), pv[...][0], jnp.float32)

    def _z(v, _):
        mloc[pl.ds(v * 16, 16)] = zf
        dloc[pl.ds(v * 16, 16)] = zf
        nloc[pl.ds(v * 16, 16)] = zf
        repv[pl.ds(v * 16, 16)] = zi
        return 0
    lax.fori_loop(0, DELTA // 16, _z, 0)

    def _scan_batches(body):
        def _issue(b, par):
            sm = _semb[par]
            pltpu.async_copy(src_hbm.at[pl.ds(b * BATCH, BATCH)],
                             _srcb[par], sm)
            pltpu.async_copy(dst_hbm.at[pl.ds(b * BATCH, BATCH)],
                             _dstb[par], sm)

        def _wait(par):
            sm = _semb[par]
            pltpu.make_async_copy(src_hbm.at[pl.ds(0, BATCH)],
                                  _srcb[par], sm).wait()
            pltpu.make_async_copy(dst_hbm.at[pl.ds(0, BATCH)],
                                  _dstb[par], sm).wait()

        _issue(0, 0)

        def _pair(bp, _):
            for par in range(2):
                b = 2 * bp + par
                nxt = b + 1

                @pl.when(nxt < NB)
                def _():
                    _issue(nxt, 1 - par)
                _wait(par)
                body(_srcb[par], _dstb[par], b)
            return 0
        lax.fori_loop(0, NB // 2, _pair, 0)

    def _p1(sr, dr, b):
        def _v(v, _):
            for u in range(5):
                vv = v * 5 + u
                sv = sr[pl.ds(vv * 16, 16)]
                dv = dr[pl.ds(vv * 16, 16)]
                dl = dv - base
                msk = (dl >= 0) & (dl < DELTA)
                plsc.store_scatter(repv, [dl], sv, mask=msk)
            return 0
        lax.fori_loop(0, VPB // 5, _v, 0)
    _scan_batches(_p1)

    def _p1b(j, _):
        rs = repv[pl.ds(j * 16, 16)]
        ss = plsc.load_gather(sfull, [rs])
        sd = sfull[pl.ds(base + j * 16, 16)]
        t = ss + sd
        e = a2v * jnp.maximum(t, t * NEG)
        mloc[pl.ds(j * 16, 16)] = e
        return 0
    lax.fori_loop(0, DELTA // 16, _p1b, 0)

    def _p2(sr, dr, b):
        def _v(v, _):
            for u in range(5):
                vv = v * 5 + u
                sv = sr[pl.ds(vv * 16, 16)]
                dv = dr[pl.ds(vv * 16, 16)]
                ss = plsc.load_gather(sfull, [sv])
                sd = plsc.load_gather(sfull, [dv])
                t = ss + sd
                e = a2v * jnp.maximum(t, t * NEG)
                dl = dv - base
                msk = (dl >= 0) & (dl < DELTA)
                mv = plsc.load_gather(mloc, [dl], mask=msk)
                w = jnp.exp(e - mv)
                plsc.addupdate_scatter(dloc, [dl], w, mask=msk)
                plsc.addupdate_scatter(nloc, [dl], w * ss, mask=msk)
            return 0
        lax.fori_loop(0, VPB // 5, _v, 0)
    _scan_batches(_p2)

    def _p3(j, _):
        d = dloc[pl.ds(j * 16, 16)] + 1e-16
        n = nloc[pl.ds(j * 16, 16)]
        o = n / d
        outloc[pl.ds(j * 16, 16)] = 1.0 / (1.0 + jnp.exp(-o))
        return 0
    lax.fori_loop(0, DELTA // 16, _p3, 0)
    pltpu.sync_copy(outloc, o_hbm.at[pl.ds(base, DELTA)])


def _sc_mesh():
    return plsc.VectorSubcoreMesh(core_axis_name="c", subcore_axis_name="s",
                                  num_cores=NC, num_subcores=NS)


def _layer1(h, src, dst, a1, w2col):
    return pl.kernel(
        _l1_body,
        out_type=jax.ShapeDtypeStruct((NPAD,), jnp.float32),
        mesh=_sc_mesh(),
        compiler_params=pltpu.CompilerParams(needs_layout_passes=False),
        scratch_types=[
            pltpu.VMEM((DELTA + 1, D), jnp.float32),
            pltpu.VMEM((DELTA + 16,), jnp.float32),
            pltpu.VMEM((DELTA + 16,), jnp.float32),
            pltpu.VMEM((D,), jnp.float32),
            pltpu.VMEM((D,), jnp.float32),
            pltpu.VMEM((BATCH,), jnp.int32),
            pltpu.VMEM((BATCH,), jnp.int32),
            pltpu.VMEM((BATCH,), jnp.int32),
            pltpu.VMEM((BATCH,), jnp.int32),
            pltpu.VMEM((BATCH + 64,), jnp.int32),
            pltpu.VMEM((BATCH + 64,), jnp.int32),
            pltpu.VMEM((DELTA,), jnp.int32),
            pltpu.VMEM((GCH, D), jnp.float32),
            pltpu.VMEM((GCH, D), jnp.float32),
            pltpu.VMEM((GCH, D), jnp.float32),
            pltpu.VMEM((GCH, D), jnp.float32),
            pltpu.VMEM((DELTA,), jnp.float32),
            pltpu.SemaphoreType.DMA,
            pltpu.SemaphoreType.DMA,
            pltpu.SemaphoreType.DMA,
            pltpu.SemaphoreType.DMA,
        ],
    )(h, src, dst, a1, w2col)


def _layer2(s, src, dst, params):
    return pl.kernel(
        _l2_body,
        out_type=jax.ShapeDtypeStruct((NPAD,), jnp.float32),
        mesh=_sc_mesh(),
        compiler_params=pltpu.CompilerParams(needs_layout_passes=False),
        scratch_types=[
            pltpu.VMEM((NPAD,), jnp.float32),
            pltpu.VMEM((DELTA,), jnp.float32),
            pltpu.VMEM((DELTA,), jnp.float32),
            pltpu.VMEM((DELTA,), jnp.float32),
            pltpu.VMEM((DELTA,), jnp.int32),
            pltpu.VMEM((BATCH,), jnp.int32),
            pltpu.VMEM((BATCH,), jnp.int32),
            pltpu.VMEM((BATCH,), jnp.int32),
            pltpu.VMEM((BATCH,), jnp.int32),
            pltpu.VMEM((16,), jnp.float32),
            pltpu.VMEM((DELTA,), jnp.float32),
            pltpu.SemaphoreType.DMA,
            pltpu.SemaphoreType.DMA,
        ],
    )(s, src, dst, params)


def kernel(feature, edge_index, W1, a1, W2, a2):
    src = edge_index[0].astype(jnp.int32)
    dst = edge_index[1].astype(jnp.int32)
    xpad = jnp.pad(feature, ((0, NPAD - N), (0, 0)))
    h = _matmul(xpad, W1)
    s = _layer1(h, src, dst, a1, W2[:, 0])
    params = jnp.pad(a2, (0, 15))
    out = _layer2(s, src, dst, params)
    return out[:N][:, None]


# speedup vs baseline: 1.7088x; 1.0190x over previous
import jax
import jax.numpy as jnp
from jax import lax
from jax.experimental import pallas as pl
from jax.experimental.pallas import tpu as pltpu
from jax.experimental.pallas import tpu_sc as plsc

N = 10000
E = 160000
D = 256
KV = D // 16
NC, NS = 2, 16
NW = NC * NS
DELTA = 320
NPAD = NW * DELTA
BATCH = 1600
NB = E // BATCH
VPB = BATCH // 16
GCH = 32
GU = 8
NEG = 0.2


def _mm_body(x_ref, w_ref, o_ref):
    o_ref[...] = jnp.dot(x_ref[...], w_ref[...],
                         preferred_element_type=jnp.float32)


def _matmul(x, w):
    m, k = x.shape
    n = w.shape[1]
    bm = 1024
    return pl.pallas_call(
        _mm_body,
        grid=(m // bm,),
        in_specs=[
            pl.BlockSpec((bm, k), lambda i: (i, 0)),
            pl.BlockSpec((k, n), lambda i: (0, 0)),
        ],
        out_specs=pl.BlockSpec((bm, n), lambda i: (i, 0)),
        out_shape=jax.ShapeDtypeStruct((m, n), jnp.float32),
    )(x, w)


def _l1_body(h_hbm, src_hbm, dst_hbm, a1_hbm, w2_hbm, s_hbm,
             acc, mflat, dflat, a1v, w2v, srcbA, srcbB, dstbA, dstbB,
             osrc, odst, repv, hsA, hsB, hdA, hdB, sloc,
             sembA, sembB, semcA, semcB):
    _srcb = [srcbA, srcbB]
    _dstb = [dstbA, dstbB]
    _hs = [hsA, hsB]
    _hd = [hdA, hdB]
    _semb = [sembA, sembB]
    _semc = [semcA, semcB]
    wid = lax.axis_index("s") * NC + lax.axis_index("c")
    base = wid * DELTA
    zf = jnp.zeros((16,), jnp.float32)
    zi = jnp.zeros((16,), jnp.int32)
    lane0 = lax.iota(jnp.int32, 16) == 0

    pltpu.sync_copy(a1_hbm, a1v)
    pltpu.sync_copy(w2_hbm, w2v)

    def _z_rows(j, _):
        for k in range(KV):
            acc[j, pl.ds(k * 16, 16)] = zf
        return 0
    lax.fori_loop(0, DELTA, _z_rows, 0)

    def _z_rep(v, _):
        repv[pl.ds(v * 16, 16)] = zi
        dflat[pl.ds(v * 16, 16)] = zf
        return 0
    lax.fori_loop(0, DELTA // 16, _z_rep, 0)

    def _scan_batches(body, init):
        def _issue(b, par):
            sm = _semb[par]
            pltpu.async_copy(src_hbm.at[pl.ds(b * BATCH, BATCH)],
                             _srcb[par], sm)
            pltpu.async_copy(dst_hbm.at[pl.ds(b * BATCH, BATCH)],
                             _dstb[par], sm)

        def _wait(par):
            sm = _semb[par]
            pltpu.make_async_copy(src_hbm.at[pl.ds(0, BATCH)],
                                  _srcb[par], sm).wait()
            pltpu.make_async_copy(dst_hbm.at[pl.ds(0, BATCH)],
                                  _dstb[par], sm).wait()

        _issue(0, 0)

        def _pair(bp, carry):
            for par in range(2):
                b = 2 * bp + par
                nxt = b + 1

                @pl.when(nxt < NB)
                def _():
                    _issue(nxt, 1 - par)
                _wait(par)
                carry = body(_srcb[par], _dstb[par], b, carry)
            return carry
        return lax.fori_loop(0, NB // 2, _pair, init)

    def _p1(sr, dr, b, carry):
        def _v(v, _):
            sv = sr[pl.ds(v * 16, 16)]
            dv = dr[pl.ds(v * 16, 16)]
            dl = dv - base
            msk = (dl >= 0) & (dl < DELTA)
            plsc.store_scatter(repv, [dl], sv, mask=msk)
            return 0
        lax.fori_loop(0, VPB, _v, 0)
        return carry
    _scan_batches(_p1, jnp.int32(0))

    a1list = [a1v[pl.ds(k * 16, 16)] for k in range(KV)]
    _gdn = lax.GatherDimensionNumbers(offset_dims=(),
                                      collapsed_slice_dims=(0,),
                                      start_index_map=(0,))
    fold_idx = [(lax.iota(jnp.int32, 16) ^ k)[:, None] for k in (8, 4, 2, 1)]

    def _allreduce(v):
        for i in fold_idx:
            v = v + lax.gather(v, i, _gdn, (1,),
                               mode=lax.GatherScatterMode.PROMISE_IN_BOUNDS)
        return v

    def _escore(hsb, hdb, i):
        ev = [zf, zf, zf, zf]
        for k in range(KV):
            hk = hsb[i, pl.ds(k * 16, 16)]
            t = hk + hdb[i, pl.ds(k * 16, 16)]
            t = jnp.maximum(t, t * NEG)
            ev[k % 4] = ev[k % 4] + a1list[k] * t
        return (ev[0] + ev[1]) + (ev[2] + ev[3])

    def _p1b(c, _):
        pltpu.sync_copy(h_hbm.at[pl.ds(base + c * GCH, GCH)], hdA)
        pltpu.async_copy(h_hbm.at[repv.at[pl.ds(c * GCH, GCH)]],
                         hsA, semcA).wait()

        def _i(i, _):
            ev = _allreduce(_escore(hsA, hdA, i))
            plsc.store_scatter(mflat, [jnp.full((16,), c * GCH + i, jnp.int32)],
                               ev, mask=lane0)
            return 0
        lax.fori_loop(0, GCH, _i, 0)
        return 0
    lax.fori_loop(0, DELTA // GCH, _p1b, 0)

    def _issue_ch(c, par):
        sm = _semc[par]
        pltpu.async_copy(h_hbm.at[osrc.at[pl.ds(c * GCH, GCH)]],
                         _hs[par], sm)
        pltpu.async_copy(h_hbm.at[odst.at[pl.ds(c * GCH, GCH)]],
                         _hd[par], sm)

    def _wait_ch(par):
        sm = _semc[par]
        pltpu.make_async_copy(h_hbm.at[pl.ds(0, GCH)], _hs[par], sm).wait()
        pltpu.make_async_copy(h_hbm.at[pl.ds(0, GCH)], _hd[par], sm).wait()

    def _proc(c, par):
        _wait_ch(par)

        def _grp(g, _):
            dlv = odst[pl.ds(c * GCH + g * GU, 16)] - base
            for u in range(GU):
                i = g * GU + u
                dl = dlv[u]
                idxv = jnp.full((16,), dl, jnp.int32)
                ev = _allreduce(_escore(_hs[par], _hd[par], i))
                mv = plsc.load_gather(mflat, [idxv])
                w = jnp.exp(ev - mv)
                plsc.addupdate_scatter(dflat, [idxv], w, mask=lane0)
                for k in range(KV):
                    plsc.addupdate(acc.at[dl, pl.ds(k * 16, 16)],
                                   w * _hs[par][i, pl.ds(k * 16, 16)])
            return 0
        lax.fori_loop(0, GCH // GU, _grp, 0)

    def _run_chunks(nch):
        def _chpair(cp, _):
            for par in range(2):
                c = 2 * cp + par

                @pl.when(c < nch)
                def _():
                    @pl.when(c + 1 < nch)
                    def _():
                        _issue_ch(c + 1, 1 - par)
                    _proc(c, par)
            return 0
        lax.fori_loop(0, (nch + 1) // 2, _chpair, 0)

    def _p2(sr, dr, b, lo):
        def _cmp(v, off):
            sv = sr[pl.ds(v * 16, 16)]
            dv = dr[pl.ds(v * 16, 16)]
            dl = dv - base
            msk = (dl >= 0) & (dl < DELTA)
            mi = msk.astype(jnp.int32)
            pos = off + plsc.cumsum(mi) - 1
            plsc.store_scatter(osrc, [pos], sv, mask=msk)
            plsc.store_scatter(odst, [pos], dv, mask=msk)
            return pos[15] + 1
        cnt1 = lax.fori_loop(0, VPB // 2, _cmp, lo)
        pre = cnt1 >= GCH

        @pl.when(pre)
        def _():
            _issue_ch(0, 0)
        cnt = lax.fori_loop(VPB // 2, VPB, _cmp, cnt1)

        nfull = cnt // GCH

        @pl.when(nfull > 0)
        def _():
            @pl.when(jnp.logical_not(pre))
            def _():
                _issue_ch(0, 0)
            _run_chunks(nfull)
            tail = nfull * GCH
            d0 = odst[pl.ds(tail, 16)]
            d1 = odst[pl.ds(tail + 16, 16)]
            s0 = osrc[pl.ds(tail, 16)]
            s1 = osrc[pl.ds(tail + 16, 16)]
            odst[pl.ds(0, 16)] = d0
            odst[pl.ds(16, 16)] = d1
            osrc[pl.ds(0, 16)] = s0
            osrc[pl.ds(16, 16)] = s1
        return cnt - nfull * GCH

    rem = _scan_batches(_p2, jnp.int32(0))

    @pl.when(rem > 0)
    def _():
        iota16 = lax.iota(jnp.int32, 16)
        for t in range(2):
            pos = rem + t * 16 + iota16
            pmsk = pos < (BATCH + 64)
            plsc.store_scatter(odst, [pos],
                               jnp.full((16,), base + DELTA, jnp.int32),
                               mask=pmsk)
            plsc.store_scatter(osrc, [pos], zi, mask=pmsk)
        _issue_ch(0, 0)
        _proc(0, 0)

    def _p3(jj, _):
        dv = plsc.load_gather(dflat, [jnp.full((16,), jj, jnp.int32)]) + 1e-16
        sv2 = zf
        for k in range(KV):
            t = acc[jj, pl.ds(k * 16, 16)] / dv
            t = jnp.maximum(t, 0.0)
            sv2 = sv2 + t * w2v[pl.ds(k * 16, 16)]
        plsc.store_scatter(sloc, [jnp.full((16,), jj, jnp.int32)],
                           jnp.full((16,), jnp.sum(sv2), jnp.float32),
                           mask=lane0)
        return 0
    lax.fori_loop(0, DELTA, _p3, 0)
    pltpu.sync_copy(sloc, s_hbm.at[pl.ds(base, DELTA)])


def _l2_body(s_hbm, src_hbm, dst_hbm, p_hbm, o_hbm,
             sfull, mloc, dloc, nloc, repv, srcbA, srcbB, dstbA, dstbB,
             pv, outloc, sembA, sembB):
    _srcb = [srcbA, srcbB]
    _dstb = [dstbA, dstbB]
    _semb = [sembA, sembB]
    wid = lax.axis_index("s") * NC + lax.axis_index("c")
    base = wid * DELTA
    zf = jnp.zeros((16,), jnp.float32)
    zi = jnp.zeros((16,), jnp.int32)

    pltpu.sync_copy(s_hbm, sfull)
    pltpu.sync_copy(p_hbm, pv)
    a2v = jnp.full((16,), pv[...][0], jnp.float32)

    def _z(v, _):
        mloc[pl.ds(v * 16, 16)] = zf
        dloc[pl.ds(v * 16, 16)] = zf
        nloc[pl.ds(v * 16, 16)] = zf
        repv[pl.ds(v * 16, 16)] = zi
        return 0
    lax.fori_loop(0, DELTA // 16, _z, 0)

    def _scan_batches(body):
        def _issue(b, par):
            sm = _semb[par]
            pltpu.async_copy(src_hbm.at[pl.ds(b * BATCH, BATCH)],
                             _srcb[par], sm)
            pltpu.async_copy(dst_hbm.at[pl.ds(b * BATCH, BATCH)],
                             _dstb[par], sm)

        def _wait(par):
            sm = _semb[par]
            pltpu.make_async_copy(src_hbm.at[pl.ds(0, BATCH)],
                                  _srcb[par], sm).wait()
            pltpu.make_async_copy(dst_hbm.at[pl.ds(0, BATCH)],
                                  _dstb[par], sm).wait()

        _issue(0, 0)

        def _pair(bp, _):
            for par in range(2):
                b = 2 * bp + par
                nxt = b + 1

                @pl.when(nxt < NB)
                def _():
                    _issue(nxt, 1 - par)
                _wait(par)
                body(_srcb[par], _dstb[par], b)
            return 0
        lax.fori_loop(0, NB // 2, _pair, 0)

    def _p1(sr, dr, b):
        def _v(v, _):
            for u in range(5):
                vv = v * 5 + u
                sv = sr[pl.ds(vv * 16, 16)]
                dv = dr[pl.ds(vv * 16, 16)]
                dl = dv - base
                msk = (dl >= 0) & (dl < DELTA)
                plsc.store_scatter(repv, [dl], sv, mask=msk)
            return 0
        lax.fori_loop(0, VPB // 5, _v, 0)
    _scan_batches(_p1)

    def _p1b(j, _):
        rs = repv[pl.ds(j * 16, 16)]
        ss = plsc.load_gather(sfull, [rs])
        sd = sfull[pl.ds(base + j * 16, 16)]
        t = ss + sd
        e = a2v * jnp.maximum(t, t * NEG)
        mloc[pl.ds(j * 16, 16)] = e
        return 0
    lax.fori_loop(0, DELTA // 16, _p1b, 0)

    def _p2(sr, dr, b):
        def _v(v, _):
            for u in range(5):
                vv = v * 5 + u
                sv = sr[pl.ds(vv * 16, 16)]
                dv = dr[pl.ds(vv * 16, 16)]
                ss = plsc.load_gather(sfull, [sv])
                sd = plsc.load_gather(sfull, [dv])
                t = ss + sd
                e = a2v * jnp.maximum(t, t * NEG)
                dl = dv - base
                msk = (dl >= 0) & (dl < DELTA)
                mv = plsc.load_gather(mloc, [dl], mask=msk)
                w = jnp.exp(e - mv)
                plsc.addupdate_scatter(dloc, [dl], w, mask=msk)
                plsc.addupdate_scatter(nloc, [dl], w * ss, mask=msk)
            return 0
        lax.fori_loop(0, VPB // 5, _v, 0)
    _scan_batches(_p2)

    def _p3(j, _):
        d = dloc[pl.ds(j * 16, 16)] + 1e-16
        n = nloc[pl.ds(j * 16, 16)]
        o = n / d
        outloc[pl.ds(j * 16, 16)] = 1.0 / (1.0 + jnp.exp(-o))
        return 0
    lax.fori_loop(0, DELTA // 16, _p3, 0)
    pltpu.sync_copy(outloc, o_hbm.at[pl.ds(base, DELTA)])


def _sc_mesh():
    return plsc.VectorSubcoreMesh(core_axis_name="c", subcore_axis_name="s",
                                  num_cores=NC, num_subcores=NS)


def _layer1(h, src, dst, a1, w2col):
    return pl.kernel(
        _l1_body,
        out_type=jax.ShapeDtypeStruct((NPAD,), jnp.float32),
        mesh=_sc_mesh(),
        compiler_params=pltpu.CompilerParams(needs_layout_passes=False),
        scratch_types=[
            pltpu.VMEM((DELTA + 1, D), jnp.float32),
            pltpu.VMEM((DELTA + 16,), jnp.float32),
            pltpu.VMEM((DELTA + 16,), jnp.float32),
            pltpu.VMEM((D,), jnp.float32),
            pltpu.VMEM((D,), jnp.float32),
            pltpu.VMEM((BATCH,), jnp.int32),
            pltpu.VMEM((BATCH,), jnp.int32),
            pltpu.VMEM((BATCH,), jnp.int32),
            pltpu.VMEM((BATCH,), jnp.int32),
            pltpu.VMEM((BATCH + 64,), jnp.int32),
            pltpu.VMEM((BATCH + 64,), jnp.int32),
            pltpu.VMEM((DELTA,), jnp.int32),
            pltpu.VMEM((GCH, D), jnp.float32),
            pltpu.VMEM((GCH, D), jnp.float32),
            pltpu.VMEM((GCH, D), jnp.float32),
            pltpu.VMEM((GCH, D), jnp.float32),
            pltpu.VMEM((DELTA,), jnp.float32),
            pltpu.SemaphoreType.DMA,
            pltpu.SemaphoreType.DMA,
            pltpu.SemaphoreType.DMA,
            pltpu.SemaphoreType.DMA,
        ],
    )(h, src, dst, a1, w2col)


def _layer2(s, src, dst, params):
    return pl.kernel(
        _l2_body,
        out_type=jax.ShapeDtypeStruct((NPAD,), jnp.float32),
        mesh=_sc_mesh(),
        compiler_params=pltpu.CompilerParams(needs_layout_passes=False),
        scratch_types=[
            pltpu.VMEM((NPAD,), jnp.float32),
            pltpu.VMEM((DELTA,), jnp.float32),
            pltpu.VMEM((DELTA,), jnp.float32),
            pltpu.VMEM((DELTA,), jnp.float32),
            pltpu.VMEM((DELTA,), jnp.int32),
            pltpu.VMEM((BATCH,), jnp.int32),
            pltpu.VMEM((BATCH,), jnp.int32),
            pltpu.VMEM((BATCH,), jnp.int32),
            pltpu.VMEM((BATCH,), jnp.int32),
            pltpu.VMEM((16,), jnp.float32),
            pltpu.VMEM((DELTA,), jnp.float32),
            pltpu.SemaphoreType.DMA,
            pltpu.SemaphoreType.DMA,
        ],
    )(s, src, dst, params)


def kernel(feature, edge_index, W1, a1, W2, a2):
    src = edge_index[0].astype(jnp.int32)
    dst = edge_index[1].astype(jnp.int32)
    xpad = jnp.pad(feature, ((0, NPAD - N), (0, 0)))
    h = _matmul(xpad, W1)
    s = _layer1(h, src, dst, a1, W2[:, 0])
    params = jnp.pad(a2, (0, 15))
    out = _layer2(s, src, dst, params)
    return out[:N][:, None]


# speedup vs baseline: 2.1082x; 1.2337x over previous
import jax
import jax.numpy as jnp
from jax import lax
from jax.experimental import pallas as pl
from jax.experimental.pallas import tpu as pltpu
from jax.experimental.pallas import tpu_sc as plsc

N = 10000
E = 160000
D = 256
KV = D // 16
NC, NS = 2, 16
NW = NC * NS
DELTA = 320
NPAD = NW * DELTA
BATCH = 1600
NB = E // BATCH
VPB = BATCH // 16
GCH = 32
GU = 8
NEG = 0.2


def _mm_body(x_ref, w_ref, o_ref):
    o_ref[...] = jnp.dot(x_ref[...], w_ref[...],
                         preferred_element_type=jnp.float32)


def _matmul(x, w):
    m, k = x.shape
    n = w.shape[1]
    bm = 1024
    return pl.pallas_call(
        _mm_body,
        grid=(m // bm,),
        in_specs=[
            pl.BlockSpec((bm, k), lambda i: (i, 0)),
            pl.BlockSpec((k, n), lambda i: (0, 0)),
        ],
        out_specs=pl.BlockSpec((bm, n), lambda i: (i, 0)),
        out_shape=jax.ShapeDtypeStruct((m, n), jnp.float32),
    )(x, w)


def _l1_body(h_hbm, src_hbm, dst_hbm, a1_hbm, w2_hbm, s_hbm,
             acc, mflat, dflat, a1v, w2v, srcbA, srcbB, dstbA, dstbB,
             osrc, odst, repv, hsA, hsB, hdA, hdB, sloc,
             sembA, sembB, semcA, semcB):
    _srcb = [srcbA, srcbB]
    _dstb = [dstbA, dstbB]
    _hs = [hsA, hsB]
    _hd = [hdA, hdB]
    _semb = [sembA, sembB]
    _semc = [semcA, semcB]
    wid = lax.axis_index("s") * NC + lax.axis_index("c")
    base = wid * DELTA
    zf = jnp.zeros((16,), jnp.float32)
    zi = jnp.zeros((16,), jnp.int32)
    lane0 = lax.iota(jnp.int32, 16) == 0

    pltpu.sync_copy(a1_hbm, a1v)
    pltpu.sync_copy(w2_hbm, w2v)

    def _z_rows(j, _):
        for k in range(KV):
            acc[j, pl.ds(k * 16, 16)] = zf
        return 0
    lax.fori_loop(0, DELTA, _z_rows, 0)

    def _z_rep(v, _):
        repv[pl.ds(v * 16, 16)] = zi
        dflat[pl.ds(v * 16, 16)] = zf
        return 0
    lax.fori_loop(0, DELTA // 16, _z_rep, 0)

    def _scan_batches(body, init):
        def _issue(b, par):
            sm = _semb[par]
            pltpu.async_copy(src_hbm.at[pl.ds(b * BATCH, BATCH)],
                             _srcb[par], sm)
            pltpu.async_copy(dst_hbm.at[pl.ds(b * BATCH, BATCH)],
                             _dstb[par], sm)

        def _wait(par):
            sm = _semb[par]
            pltpu.make_async_copy(src_hbm.at[pl.ds(0, BATCH)],
                                  _srcb[par], sm).wait()
            pltpu.make_async_copy(dst_hbm.at[pl.ds(0, BATCH)],
                                  _dstb[par], sm).wait()

        _issue(0, 0)

        def _pair(bp, carry):
            for par in range(2):
                b = 2 * bp + par
                nxt = b + 1

                @pl.when(nxt < NB)
                def _():
                    _issue(nxt, 1 - par)
                _wait(par)
                carry = body(_srcb[par], _dstb[par], b, carry)
            return carry
        return lax.fori_loop(0, NB // 2, _pair, init)

    def _p1(sr, dr, b, carry):
        def _v(v, _):
            sv = sr[pl.ds(v * 16, 16)]
            dv = dr[pl.ds(v * 16, 16)]
            dl = dv - base
            msk = (dl >= 0) & (dl < DELTA)
            plsc.store_scatter(repv, [dl], sv, mask=msk)
            return 0
        lax.fori_loop(0, VPB, _v, 0)
        return carry
    _scan_batches(_p1, jnp.int32(0))

    a1list = [a1v[pl.ds(k * 16, 16)] for k in range(KV)]
    _gdn = lax.GatherDimensionNumbers(offset_dims=(),
                                      collapsed_slice_dims=(0,),
                                      start_index_map=(0,))
    fold_idx = [(lax.iota(jnp.int32, 16) ^ k)[:, None] for k in (8, 4, 2, 1)]

    def _allreduce(v):
        for i in fold_idx:
            v = v + lax.gather(v, i, _gdn, (1,),
                               mode=lax.GatherScatterMode.PROMISE_IN_BOUNDS)
        return v

    def _escore(hsb, hdb, i):
        ev = [zf, zf, zf, zf]
        for k in range(KV):
            hk = hsb[i, pl.ds(k * 16, 16)]
            t = hk + hdb[i, pl.ds(k * 16, 16)]
            t = jnp.maximum(t, t * NEG)
            ev[k % 4] = ev[k % 4] + a1list[k] * t
        return (ev[0] + ev[1]) + (ev[2] + ev[3])

    def _p1b(c, _):
        pltpu.sync_copy(h_hbm.at[pl.ds(base + c * GCH, GCH)], hdA)
        pltpu.async_copy(h_hbm.at[repv.at[pl.ds(c * GCH, GCH)]],
                         hsA, semcA).wait()

        def _i(i, _):
            ev = _allreduce(_escore(hsA, hdA, i))
            plsc.store_scatter(mflat, [jnp.full((16,), c * GCH + i, jnp.int32)],
                               ev, mask=lane0)
            return 0
        lax.fori_loop(0, GCH, _i, 0)
        return 0
    lax.fori_loop(0, DELTA // GCH, _p1b, 0)

    def _issue_ch(c, par):
        sm = _semc[par]
        pltpu.async_copy(h_hbm.at[osrc.at[pl.ds(c * GCH, GCH)]],
                         _hs[par], sm)
        pltpu.async_copy(h_hbm.at[odst.at[pl.ds(c * GCH, GCH)]],
                         _hd[par], sm)

    def _wait_ch(par):
        sm = _semc[par]
        pltpu.make_async_copy(h_hbm.at[pl.ds(0, GCH)], _hs[par], sm).wait()
        pltpu.make_async_copy(h_hbm.at[pl.ds(0, GCH)], _hd[par], sm).wait()

    def _proc(c, par):
        _wait_ch(par)

        def _grp(g, _):
            dlv = odst[pl.ds(c * GCH + g * GU, 16)] - base
            for u in range(GU):
                i = g * GU + u
                dl = dlv[u]
                idxv = jnp.full((16,), dl, jnp.int32)
                ev = _allreduce(_escore(_hs[par], _hd[par], i))
                mv = plsc.load_gather(mflat, [idxv])
                w = jnp.exp(ev - mv)
                plsc.addupdate_scatter(dflat, [idxv], w, mask=lane0)
                prods = [w * _hs[par][i, pl.ds(k * 16, 16)]
                         for k in range(KV)]
                for k in range(KV):
                    plsc.addupdate(acc.at[dl, pl.ds(k * 16, 16)], prods[k])
            return 0
        lax.fori_loop(0, GCH // GU, _grp, 0)

    def _run_chunks(nch):
        def _chpair(cp, _):
            for par in range(2):
                c = 2 * cp + par

                @pl.when(c < nch)
                def _():
                    @pl.when(c + 1 < nch)
                    def _():
                        _issue_ch(c + 1, 1 - par)
                    _proc(c, par)
            return 0
        lax.fori_loop(0, (nch + 1) // 2, _chpair, 0)

    def _p2(sr, dr, b, lo):
        def _cmp(v, off):
            sv = sr[pl.ds(v * 16, 16)]
            dv = dr[pl.ds(v * 16, 16)]
            dl = dv - base
            msk = (dl >= 0) & (dl < DELTA)
            mi = msk.astype(jnp.int32)
            pos = off + plsc.cumsum(mi) - 1
            plsc.store_scatter(osrc, [pos], sv, mask=msk)
            plsc.store_scatter(odst, [pos], dv, mask=msk)
            return pos[15] + 1
        cnt1 = lax.fori_loop(0, VPB // 2, _cmp, lo)
        pre = cnt1 >= GCH

        @pl.when(pre)
        def _():
            _issue_ch(0, 0)
        cnt = lax.fori_loop(VPB // 2, VPB, _cmp, cnt1)

        nfull = cnt // GCH

        @pl.when(nfull > 0)
        def _():
            @pl.when(jnp.logical_not(pre))
            def _():
                _issue_ch(0, 0)
            _run_chunks(nfull)
            tail = nfull * GCH
            d0 = odst[pl.ds(tail, 16)]
            d1 = odst[pl.ds(tail + 16, 16)]
            s0 = osrc[pl.ds(tail, 16)]
            s1 = osrc[pl.ds(tail + 16, 16)]
            odst[pl.ds(0, 16)] = d0
            odst[pl.ds(16, 16)] = d1
            osrc[pl.ds(0, 16)] = s0
            osrc[pl.ds(16, 16)] = s1
        return cnt - nfull * GCH

    rem = _scan_batches(_p2, jnp.int32(0))

    @pl.when(rem > 0)
    def _():
        iota16 = lax.iota(jnp.int32, 16)
        for t in range(2):
            pos = rem + t * 16 + iota16
            pmsk = pos < (BATCH + 64)
            plsc.store_scatter(odst, [pos],
                               jnp.full((16,), base + DELTA, jnp.int32),
                               mask=pmsk)
            plsc.store_scatter(osrc, [pos], zi, mask=pmsk)
        _issue_ch(0, 0)
        _proc(0, 0)

    def _p3(jj, _):
        dv = plsc.load_gather(dflat, [jnp.full((16,), jj, jnp.int32)]) + 1e-16
        sv2 = zf
        for k in range(KV):
            t = acc[jj, pl.ds(k * 16, 16)] / dv
            t = jnp.maximum(t, 0.0)
            sv2 = sv2 + t * w2v[pl.ds(k * 16, 16)]
        plsc.store_scatter(sloc, [jnp.full((16,), jj, jnp.int32)],
                           jnp.full((16,), jnp.sum(sv2), jnp.float32),
                           mask=lane0)
        return 0
    lax.fori_loop(0, DELTA, _p3, 0)
    pltpu.sync_copy(sloc, s_hbm.at[pl.ds(base, DELTA)])


def _l2_body(s_hbm, src_hbm, dst_hbm, p_hbm, o_hbm,
             sfull, mloc, dloc, nloc, repv, srcbA, srcbB, dstbA, dstbB,
             pv, outloc, sembA, sembB):
    _srcb = [srcbA, srcbB]
    _dstb = [dstbA, dstbB]
    _semb = [sembA, sembB]
    wid = lax.axis_index("s") * NC + lax.axis_index("c")
    base = wid * DELTA
    zf = jnp.zeros((16,), jnp.float32)
    zi = jnp.zeros((16,), jnp.int32)

    pltpu.sync_copy(s_hbm, sfull)
    pltpu.sync_copy(p_hbm, pv)
    a2v = jnp.full((16,), pv[...][0], jnp.float32)

    def _z(v, _):
        mloc[pl.ds(v * 16, 16)] = zf
        dloc[pl.ds(v * 16, 16)] = zf
        nloc[pl.ds(v * 16, 16)] = zf
        repv[pl.ds(v * 16, 16)] = zi
        return 0
    lax.fori_loop(0, DELTA // 16, _z, 0)

    def _scan_batches(body):
        def _issue(b, par):
            sm = _semb[par]
            pltpu.async_copy(src_hbm.at[pl.ds(b * BATCH, BATCH)],
                             _srcb[par], sm)
            pltpu.async_copy(dst_hbm.at[pl.ds(b * BATCH, BATCH)],
                             _dstb[par], sm)

        def _wait(par):
            sm = _semb[par]
            pltpu.make_async_copy(src_hbm.at[pl.ds(0, BATCH)],
                                  _srcb[par], sm).wait()
            pltpu.make_async_copy(dst_hbm.at[pl.ds(0, BATCH)],
                                  _dstb[par], sm).wait()

        _issue(0, 0)

        def _pair(bp, _):
            for par in range(2):
                b = 2 * bp + par
                nxt = b + 1

                @pl.when(nxt < NB)
                def _():
                    _issue(nxt, 1 - par)
                _wait(par)
                body(_srcb[par], _dstb[par], b)
            return 0
        lax.fori_loop(0, NB // 2, _pair, 0)

    def _p1(sr, dr, b):
        def _v(v, _):
            for u in range(5):
                vv = v * 5 + u
                sv = sr[pl.ds(vv * 16, 16)]
                dv = dr[pl.ds(vv * 16, 16)]
                dl = dv - base
                msk = (dl >= 0) & (dl < DELTA)
                plsc.store_scatter(repv, [dl], sv, mask=msk)
            return 0
        lax.fori_loop(0, VPB // 5, _v, 0)
    _scan_batches(_p1)

    def _p1b(j, _):
        rs = repv[pl.ds(j * 16, 16)]
        ss = plsc.load_gather(sfull, [rs])
        sd = sfull[pl.ds(base + j * 16, 16)]
        t = ss + sd
        e = a2v * jnp.maximum(t, t * NEG)
        mloc[pl.ds(j * 16, 16)] = e
        return 0
    lax.fori_loop(0, DELTA // 16, _p1b, 0)

    def _p2(sr, dr, b):
        def _v(v, _):
            for u in range(5):
                vv = v * 5 + u
                sv = sr[pl.ds(vv * 16, 16)]
                dv = dr[pl.ds(vv * 16, 16)]
                ss = plsc.load_gather(sfull, [sv])
                sd = plsc.load_gather(sfull, [dv])
                t = ss + sd
                e = a2v * jnp.maximum(t, t * NEG)
                dl = dv - base
                msk = (dl >= 0) & (dl < DELTA)
                mv = plsc.load_gather(mloc, [dl], mask=msk)
                w = jnp.exp(e - mv)
                plsc.addupdate_scatter(dloc, [dl], w, mask=msk)
                plsc.addupdate_scatter(nloc, [dl], w * ss, mask=msk)
            return 0
        lax.fori_loop(0, VPB // 5, _v, 0)
    _scan_batches(_p2)

    def _p3(j, _):
        d = dloc[pl.ds(j * 16, 16)] + 1e-16
        n = nloc[pl.ds(j * 16, 16)]
        o = n / d
        outloc[pl.ds(j * 16, 16)] = 1.0 / (1.0 + jnp.exp(-o))
        return 0
    lax.fori_loop(0, DELTA // 16, _p3, 0)
    pltpu.sync_copy(outloc, o_hbm.at[pl.ds(base, DELTA)])


def _sc_mesh():
    return plsc.VectorSubcoreMesh(core_axis_name="c", subcore_axis_name="s",
                                  num_cores=NC, num_subcores=NS)


def _layer1(h, src, dst, a1, w2col):
    return pl.kernel(
        _l1_body,
        out_type=jax.ShapeDtypeStruct((NPAD,), jnp.float32),
        mesh=_sc_mesh(),
        compiler_params=pltpu.CompilerParams(needs_layout_passes=False),
        scratch_types=[
            pltpu.VMEM((DELTA + 1, D), jnp.float32),
            pltpu.VMEM((DELTA + 16,), jnp.float32),
            pltpu.VMEM((DELTA + 16,), jnp.float32),
            pltpu.VMEM((D,), jnp.float32),
            pltpu.VMEM((D,), jnp.float32),
            pltpu.VMEM((BATCH,), jnp.int32),
            pltpu.VMEM((BATCH,), jnp.int32),
            pltpu.VMEM((BATCH,), jnp.int32),
            pltpu.VMEM((BATCH,), jnp.int32),
            pltpu.VMEM((BATCH + 64,), jnp.int32),
            pltpu.VMEM((BATCH + 64,), jnp.int32),
            pltpu.VMEM((DELTA,), jnp.int32),
            pltpu.VMEM((GCH, D), jnp.float32),
            pltpu.VMEM((GCH, D), jnp.float32),
            pltpu.VMEM((GCH, D), jnp.float32),
            pltpu.VMEM((GCH, D), jnp.float32),
            pltpu.VMEM((DELTA,), jnp.float32),
            pltpu.SemaphoreType.DMA,
            pltpu.SemaphoreType.DMA,
            pltpu.SemaphoreType.DMA,
            pltpu.SemaphoreType.DMA,
        ],
    )(h, src, dst, a1, w2col)


def _layer2(s, src, dst, params):
    return pl.kernel(
        _l2_body,
        out_type=jax.ShapeDtypeStruct((NPAD,), jnp.float32),
        mesh=_sc_mesh(),
        compiler_params=pltpu.CompilerParams(needs_layout_passes=False),
        scratch_types=[
            pltpu.VMEM((NPAD,), jnp.float32),
            pltpu.VMEM((DELTA,), jnp.float32),
            pltpu.VMEM((DELTA,), jnp.float32),
            pltpu.VMEM((DELTA,), jnp.float32),
            pltpu.VMEM((DELTA,), jnp.int32),
            pltpu.VMEM((BATCH,), jnp.int32),
            pltpu.VMEM((BATCH,), jnp.int32),
            pltpu.VMEM((BATCH,), jnp.int32),
            pltpu.VMEM((BATCH,), jnp.int32),
            pltpu.VMEM((16,), jnp.float32),
            pltpu.VMEM((DELTA,), jnp.float32),
            pltpu.SemaphoreType.DMA,
            pltpu.SemaphoreType.DMA,
        ],
    )(s, src, dst, params)


def kernel(feature, edge_index, W1, a1, W2, a2):
    src = edge_index[0].astype(jnp.int32)
    dst = edge_index[1].astype(jnp.int32)
    xpad = jnp.pad(feature, ((0, NPAD - N), (0, 0)))
    h = _matmul(xpad, W1)
    s = _layer1(h, src, dst, a1, W2[:, 0])
    params = jnp.pad(a2, (0, 15))
    out = _layer2(s, src, dst, params)
    return out[:N][:, None]


# speedup vs baseline: 2.4308x; 1.1530x over previous
import jax
import jax.numpy as jnp
from jax import lax
from jax.experimental import pallas as pl
from jax.experimental.pallas import tpu as pltpu
from jax.experimental.pallas import tpu_sc as plsc

N = 10000
E = 160000
D = 256
KV = D // 16
NC, NS = 2, 16
NW = NC * NS
DELTA = 320
NPAD = NW * DELTA
BATCH = 1600
NB = E // BATCH
VPB = BATCH // 16
GCH = 32
GU = 8
NEG = 0.2


def _mm_body(x_ref, w_ref, o_ref):
    o_ref[...] = jnp.dot(x_ref[...], w_ref[...],
                         preferred_element_type=jnp.float32)


def _matmul(x, w):
    m, k = x.shape
    n = w.shape[1]
    bm = 1024
    return pl.pallas_call(
        _mm_body,
        grid=(m // bm,),
        in_specs=[
            pl.BlockSpec((bm, k), lambda i: (i, 0)),
            pl.BlockSpec((k, n), lambda i: (0, 0)),
        ],
        out_specs=pl.BlockSpec((bm, n), lambda i: (i, 0)),
        out_shape=jax.ShapeDtypeStruct((m, n), jnp.float32),
    )(x, w)


def _l1_body(h_hbm, src_hbm, dst_hbm, a1_hbm, w2_hbm, s_hbm,
             acc, mflat, dflat, a1v, w2v, srcbA, srcbB, dstbA, dstbB,
             osrc, odst, repv, hsA, hsB, hdA, hdB, sloc,
             sembA, sembB, semcA, semcB):
    _srcb = [srcbA, srcbB]
    _dstb = [dstbA, dstbB]
    _hs = [hsA, hsB]
    _hd = [hdA, hdB]
    _semb = [sembA, sembB]
    _semc = [semcA, semcB]
    wid = lax.axis_index("s") * NC + lax.axis_index("c")
    base = wid * DELTA
    zf = jnp.zeros((16,), jnp.float32)
    zi = jnp.zeros((16,), jnp.int32)
    lane0 = lax.iota(jnp.int32, 16) == 0

    pltpu.sync_copy(a1_hbm, a1v)
    pltpu.sync_copy(w2_hbm, w2v)

    def _z_rows(j, _):
        for k in range(KV):
            acc[j, pl.ds(k * 16, 16)] = zf
        return 0
    lax.fori_loop(0, DELTA, _z_rows, 0)

    def _z_rep(v, _):
        repv[pl.ds(v * 16, 16)] = zi
        dflat[pl.ds(v * 16, 16)] = zf
        return 0
    lax.fori_loop(0, DELTA // 16, _z_rep, 0)

    def _scan_batches(body, init):
        def _issue(b, par):
            sm = _semb[par]
            pltpu.async_copy(src_hbm.at[pl.ds(b * BATCH, BATCH)],
                             _srcb[par], sm)
            pltpu.async_copy(dst_hbm.at[pl.ds(b * BATCH, BATCH)],
                             _dstb[par], sm)

        def _wait(par):
            sm = _semb[par]
            pltpu.make_async_copy(src_hbm.at[pl.ds(0, BATCH)],
                                  _srcb[par], sm).wait()
            pltpu.make_async_copy(dst_hbm.at[pl.ds(0, BATCH)],
                                  _dstb[par], sm).wait()

        _issue(0, 0)

        def _pair(bp, carry):
            for par in range(2):
                b = 2 * bp + par
                nxt = b + 1

                @pl.when(nxt < NB)
                def _():
                    _issue(nxt, 1 - par)
                _wait(par)
                carry = body(_srcb[par], _dstb[par], b, carry)
            return carry
        return lax.fori_loop(0, NB // 2, _pair, init)

    def _p1(sr, dr, b, carry):
        def _v(v, _):
            sv = sr[pl.ds(v * 16, 16)]
            dv = dr[pl.ds(v * 16, 16)]
            dl = dv - base
            msk = (dl >= 0) & (dl < DELTA)
            plsc.store_scatter(repv, [dl], sv, mask=msk)
            return 0
        lax.fori_loop(0, VPB, _v, 0)
        return carry
    _scan_batches(_p1, jnp.int32(0))

    a1list = [a1v[pl.ds(k * 16, 16)] for k in range(KV)]
    _gdn = lax.GatherDimensionNumbers(offset_dims=(),
                                      collapsed_slice_dims=(0,),
                                      start_index_map=(0,))
    fold_idx = [(lax.iota(jnp.int32, 16) ^ k)[:, None] for k in (8, 4, 2, 1)]

    def _allreduce(v):
        for i in fold_idx:
            v = v + lax.gather(v, i, _gdn, (1,),
                               mode=lax.GatherScatterMode.PROMISE_IN_BOUNDS)
        return v

    def _escore(hsb, hdb, i):
        ev = [zf, zf, zf, zf]
        for k in range(KV):
            hk = hsb[i, pl.ds(k * 16, 16)]
            t = hk + hdb[i, pl.ds(k * 16, 16)]
            t = jnp.maximum(t, t * NEG)
            ev[k % 4] = ev[k % 4] + a1list[k] * t
        return (ev[0] + ev[1]) + (ev[2] + ev[3])

    def _p1b(c, _):
        pltpu.sync_copy(h_hbm.at[pl.ds(base + c * GCH, GCH)], hdA)
        pltpu.async_copy(h_hbm.at[repv.at[pl.ds(c * GCH, GCH)]],
                         hsA, semcA).wait()

        def _i(i, _):
            ev = _allreduce(_escore(hsA, hdA, i))
            plsc.store_scatter(mflat, [jnp.full((16,), c * GCH + i, jnp.int32)],
                               ev, mask=lane0)
            return 0
        lax.fori_loop(0, GCH, _i, 0)
        return 0
    lax.fori_loop(0, DELTA // GCH, _p1b, 0)

    def _issue_ch(c, par):
        sm = _semc[par]
        pltpu.async_copy(h_hbm.at[osrc.at[pl.ds(c * GCH, GCH)]],
                         _hs[par], sm)
        pltpu.async_copy(h_hbm.at[odst.at[pl.ds(c * GCH, GCH)]],
                         _hd[par], sm)

    def _wait_ch(par):
        sm = _semc[par]
        pltpu.make_async_copy(h_hbm.at[pl.ds(0, GCH)], _hs[par], sm).wait()
        pltpu.make_async_copy(h_hbm.at[pl.ds(0, GCH)], _hd[par], sm).wait()

    def _proc(c, par):
        _wait_ch(par)

        def _grp(g, _):
            dlv = odst[pl.ds(c * GCH + g * GU, 16)] - base
            for u in range(GU):
                i = g * GU + u
                dl = dlv[u]
                idxv = jnp.full((16,), dl, jnp.int32)
                ev = _allreduce(_escore(_hs[par], _hd[par], i))
                mv = plsc.load_gather(mflat, [idxv])
                w = jnp.exp(ev - mv)
                plsc.addupdate_scatter(dflat, [idxv], w, mask=lane0)
                prods = [w * _hs[par][i, pl.ds(k * 16, 16)]
                         for k in range(KV)]
                for k in range(KV):
                    plsc.addupdate(acc.at[dl, pl.ds(k * 16, 16)], prods[k])
            return 0
        lax.fori_loop(0, GCH // GU, _grp, 0)

    def _run_chunks(nch):
        def _chpair(cp, _):
            for par in range(2):
                c = 2 * cp + par

                @pl.when(c < nch)
                def _():
                    @pl.when(c + 1 < nch)
                    def _():
                        _issue_ch(c + 1, 1 - par)
                    _proc(c, par)
            return 0
        lax.fori_loop(0, (nch + 1) // 2, _chpair, 0)

    def _p2(sr, dr, b, lo):
        def _cmp(v, off):
            sv = sr[pl.ds(v * 16, 16)]
            dv = dr[pl.ds(v * 16, 16)]
            dl = dv - base
            msk = (dl >= 0) & (dl < DELTA)
            mi = msk.astype(jnp.int32)
            pos = off + plsc.cumsum(mi) - 1
            plsc.store_scatter(osrc, [pos], sv, mask=msk)
            plsc.store_scatter(odst, [pos], dv, mask=msk)
            return pos[15] + 1
        cnt1 = lax.fori_loop(0, VPB // 2, _cmp, lo)
        pre = cnt1 >= GCH

        @pl.when(pre)
        def _():
            _issue_ch(0, 0)
        cnt = lax.fori_loop(VPB // 2, VPB, _cmp, cnt1)

        nfull = cnt // GCH

        @pl.when(nfull > 0)
        def _():
            @pl.when(jnp.logical_not(pre))
            def _():
                _issue_ch(0, 0)
            _run_chunks(nfull)
            tail = nfull * GCH
            d0 = odst[pl.ds(tail, 16)]
            d1 = odst[pl.ds(tail + 16, 16)]
            s0 = osrc[pl.ds(tail, 16)]
            s1 = osrc[pl.ds(tail + 16, 16)]
            odst[pl.ds(0, 16)] = d0
            odst[pl.ds(16, 16)] = d1
            osrc[pl.ds(0, 16)] = s0
            osrc[pl.ds(16, 16)] = s1
        return cnt - nfull * GCH

    rem = _scan_batches(_p2, jnp.int32(0))

    @pl.when(rem > 0)
    def _():
        iota16 = lax.iota(jnp.int32, 16)
        for t in range(2):
            pos = rem + t * 16 + iota16
            pmsk = pos < (BATCH + 64)
            plsc.store_scatter(odst, [pos],
                               jnp.full((16,), base + DELTA, jnp.int32),
                               mask=pmsk)
            plsc.store_scatter(osrc, [pos], zi, mask=pmsk)
        _issue_ch(0, 0)
        _proc(0, 0)

    def _p3(jj, _):
        dv = plsc.load_gather(dflat, [jnp.full((16,), jj, jnp.int32)]) + 1e-16
        sv2 = zf
        for k in range(KV):
            t = acc[jj, pl.ds(k * 16, 16)] / dv
            t = jnp.maximum(t, 0.0)
            sv2 = sv2 + t * w2v[pl.ds(k * 16, 16)]
        plsc.store_scatter(sloc, [jnp.full((16,), jj, jnp.int32)],
                           jnp.full((16,), jnp.sum(sv2), jnp.float32),
                           mask=lane0)
        return 0
    lax.fori_loop(0, DELTA, _p3, 0)
    pltpu.sync_copy(sloc, s_hbm.at[pl.ds(base, DELTA)])


def _l2_body(s_hbm, src_hbm, dst_hbm, p_hbm, o_hbm,
             sfull, mloc, dloc, nloc, repv, srcbA, srcbB, dstbA, dstbB,
             pv, outloc, sembA, sembB):
    _srcb = [srcbA, srcbB]
    _dstb = [dstbA, dstbB]
    _semb = [sembA, sembB]
    wid = lax.axis_index("s") * NC + lax.axis_index("c")
    base = wid * DELTA
    zf = jnp.zeros((16,), jnp.float32)
    zi = jnp.zeros((16,), jnp.int32)

    pltpu.sync_copy(s_hbm, sfull)
    pltpu.sync_copy(p_hbm, pv)
    a2v = jnp.full((16,), pv[...][0], jnp.float32)

    def _z(v, _):
        mloc[pl.ds(v * 16, 16)] = zf
        dloc[pl.ds(v * 16, 16)] = zf
        nloc[pl.ds(v * 16, 16)] = zf
        repv[pl.ds(v * 16, 16)] = zi
        return 0
    lax.fori_loop(0, DELTA // 16, _z, 0)

    def _scan_batches(body):
        def _issue(b, par):
            sm = _semb[par]
            pltpu.async_copy(src_hbm.at[pl.ds(b * BATCH, BATCH)],
                             _srcb[par], sm)
            pltpu.async_copy(dst_hbm.at[pl.ds(b * BATCH, BATCH)],
                             _dstb[par], sm)

        def _wait(par):
            sm = _semb[par]
            pltpu.make_async_copy(src_hbm.at[pl.ds(0, BATCH)],
                                  _srcb[par], sm).wait()
            pltpu.make_async_copy(dst_hbm.at[pl.ds(0, BATCH)],
                                  _dstb[par], sm).wait()

        _issue(0, 0)

        def _pair(bp, _):
            for par in range(2):
                b = 2 * bp + par
                nxt = b + 1

                @pl.when(nxt < NB)
                def _():
                    _issue(nxt, 1 - par)
                _wait(par)
                body(_srcb[par], _dstb[par], b)
            return 0
        lax.fori_loop(0, NB // 2, _pair, 0)

    def _p1(sr, dr, b):
        def _v(v, _):
            for u in range(5):
                vv = v * 5 + u
                sv = sr[pl.ds(vv * 16, 16)]
                dv = dr[pl.ds(vv * 16, 16)]
                dl = dv - base
                msk = (dl >= 0) & (dl < DELTA)
                plsc.store_scatter(repv, [dl], sv, mask=msk)
            return 0
        lax.fori_loop(0, VPB // 5, _v, 0)
    _scan_batches(_p1)

    def _p1b(j, _):
        rs = repv[pl.ds(j * 16, 16)]
        ss = plsc.load_gather(sfull, [rs])
        sd = sfull[pl.ds(base + j * 16, 16)]
        t = ss + sd
        e = a2v * jnp.maximum(t, t * NEG)
        mloc[pl.ds(j * 16, 16)] = e
        return 0
    lax.fori_loop(0, DELTA // 16, _p1b, 0)

    def _p2(sr, dr, b):
        def _v(v, _):
            acc2 = []
            for u in range(5):
                vv = v * 5 + u
                sv = sr[pl.ds(vv * 16, 16)]
                dv = dr[pl.ds(vv * 16, 16)]
                ss = plsc.load_gather(sfull, [sv])
                sd = plsc.load_gather(sfull, [dv])
                t = ss + sd
                e = a2v * jnp.maximum(t, t * NEG)
                dl = dv - base
                msk = (dl >= 0) & (dl < DELTA)
                mv = plsc.load_gather(mloc, [dl], mask=msk)
                w = jnp.exp(e - mv)
                acc2.append((dl, w, w * ss, msk))
            for dl, w, ws, msk in acc2:
                plsc.addupdate_scatter(dloc, [dl], w, mask=msk)
                plsc.addupdate_scatter(nloc, [dl], ws, mask=msk)
            return 0
        lax.fori_loop(0, VPB // 5, _v, 0)
    _scan_batches(_p2)

    def _p3(j, _):
        d = dloc[pl.ds(j * 16, 16)] + 1e-16
        n = nloc[pl.ds(j * 16, 16)]
        o = n / d
        outloc[pl.ds(j * 16, 16)] = 1.0 / (1.0 + jnp.exp(-o))
        return 0
    lax.fori_loop(0, DELTA // 16, _p3, 0)
    pltpu.sync_copy(outloc, o_hbm.at[pl.ds(base, DELTA)])


def _sc_mesh():
    return plsc.VectorSubcoreMesh(core_axis_name="c", subcore_axis_name="s",
                                  num_cores=NC, num_subcores=NS)


def _layer1(h, src, dst, a1, w2col):
    return pl.kernel(
        _l1_body,
        out_type=jax.ShapeDtypeStruct((NPAD,), jnp.float32),
        mesh=_sc_mesh(),
        compiler_params=pltpu.CompilerParams(needs_layout_passes=False),
        scratch_types=[
            pltpu.VMEM((DELTA + 1, D), jnp.float32),
            pltpu.VMEM((DELTA + 16,), jnp.float32),
            pltpu.VMEM((DELTA + 16,), jnp.float32),
            pltpu.VMEM((D,), jnp.float32),
            pltpu.VMEM((D,), jnp.float32),
            pltpu.VMEM((BATCH,), jnp.int32),
            pltpu.VMEM((BATCH,), jnp.int32),
            pltpu.VMEM((BATCH,), jnp.int32),
            pltpu.VMEM((BATCH,), jnp.int32),
            pltpu.VMEM((BATCH + 64,), jnp.int32),
            pltpu.VMEM((BATCH + 64,), jnp.int32),
            pltpu.VMEM((DELTA,), jnp.int32),
            pltpu.VMEM((GCH, D), jnp.float32),
            pltpu.VMEM((GCH, D), jnp.float32),
            pltpu.VMEM((GCH, D), jnp.float32),
            pltpu.VMEM((GCH, D), jnp.float32),
            pltpu.VMEM((DELTA,), jnp.float32),
            pltpu.SemaphoreType.DMA,
            pltpu.SemaphoreType.DMA,
            pltpu.SemaphoreType.DMA,
            pltpu.SemaphoreType.DMA,
        ],
    )(h, src, dst, a1, w2col)


def _layer2(s, src, dst, params):
    return pl.kernel(
        _l2_body,
        out_type=jax.ShapeDtypeStruct((NPAD,), jnp.float32),
        mesh=_sc_mesh(),
        compiler_params=pltpu.CompilerParams(needs_layout_passes=False),
        scratch_types=[
            pltpu.VMEM((NPAD,), jnp.float32),
            pltpu.VMEM((DELTA,), jnp.float32),
            pltpu.VMEM((DELTA,), jnp.float32),
            pltpu.VMEM((DELTA,), jnp.float32),
            pltpu.VMEM((DELTA,), jnp.int32),
            pltpu.VMEM((BATCH,), jnp.int32),
            pltpu.VMEM((BATCH,), jnp.int32),
            pltpu.VMEM((BATCH,), jnp.int32),
            pltpu.VMEM((BATCH,), jnp.int32),
            pltpu.VMEM((16,), jnp.float32),
            pltpu.VMEM((DELTA,), jnp.float32),
            pltpu.SemaphoreType.DMA,
            pltpu.SemaphoreType.DMA,
        ],
    )(s, src, dst, params)


def kernel(feature, edge_index, W1, a1, W2, a2):
    src = edge_index[0].astype(jnp.int32)
    dst = edge_index[1].astype(jnp.int32)
    xpad = jnp.pad(feature, ((0, NPAD - N), (0, 0)))
    h = _matmul(xpad, W1)
    s = _layer1(h, src, dst, a1, W2[:, 0])
    params = jnp.pad(a2, (0, 15))
    out = _layer2(s, src, dst, params)
    return out[:N][:, None]
